# baseline (speedup 1.0000x reference)
import functools

import jax
import jax.numpy as jnp
from jax import lax
from jax.experimental import pallas as pl
from jax.experimental.pallas import tpu as pltpu
from jax.experimental.pallas import tpu_sc as plsc

D_GLYPH = 512
D = 1024
BATCH = 4
SEQ = 2048
TOK = BATCH * SEQ
NC = 2
NS = 16
NW = NC * NS
NHALF = 2
HTOK = TOK // NHALF
TPW = HTOK // NW
CH = 16
NCH = TPW // CH
LN_EPS = 1e-12
VPG = D_GLYPH // 16


def _sc_gather_sum(ids, glyph, graph, stroke):
  mesh = plsc.VectorSubcoreMesh(core_axis_name="c", subcore_axis_name="s")

  @functools.partial(
      pl.kernel,
      mesh=mesh,
      out_type=[
          jax.ShapeDtypeStruct((HTOK, D), jnp.float32),
          jax.ShapeDtypeStruct((NW, 16), jnp.float32),
      ],
      scratch_types=[
          pltpu.VMEM((TPW,), jnp.int32),
          pltpu.VMEM((CH, D_GLYPH), jnp.float32),
          pltpu.VMEM((CH, D_GLYPH), jnp.float32),
          pltpu.VMEM((CH, D), jnp.float32),
          pltpu.VMEM((CH, D), jnp.float32),
          pltpu.VMEM((CH, D_GLYPH), jnp.float32),
          pltpu.VMEM((CH, D_GLYPH), jnp.float32),
          pltpu.VMEM((CH, D), jnp.float32),
          pltpu.VMEM((CH, D), jnp.float32),
          pltpu.VMEM((16,), jnp.float32),
          pltpu.SemaphoreType.DMA,
          pltpu.SemaphoreType.DMA,
          pltpu.SemaphoreType.DMA,
          pltpu.SemaphoreType.DMA,
      ],
  )
  def body(ids_hbm, glyph_hbm, graph_hbm, stroke_hbm, sum_hbm, sq_hbm,
           idx_all, gly0, gra0, str0, out0, gly1, gra1, str1, out1,
           sq_v, semg0, semg1, semo0, semo1):
    wid = lax.axis_index("s") * NC + lax.axis_index("c")
    base = wid * TPW
    pltpu.sync_copy(ids_hbm.at[pl.ds(base, TPW)], idx_all)
    bufs = ((gly0, gra0, str0, out0, semg0, semo0),
            (gly1, gra1, str1, out1, semg1, semo1))

    def fire(k, b):
      gly, gra, stv, _, semg, _ = bufs[b]
      idx = idx_all.at[pl.ds(k * CH, CH)]
      pltpu.async_copy(glyph_hbm.at[idx], gly, semg)
      pltpu.async_copy(graph_hbm.at[idx], gra, semg)
      pltpu.async_copy(stroke_hbm.at[idx], stv, semg)

    def wait_gathers(k, b):
      gly, gra, stv, _, semg, _ = bufs[b]
      idx = idx_all.at[pl.ds(k * CH, CH)]
      pltpu.make_async_copy(glyph_hbm.at[idx], gly, semg).wait()
      pltpu.make_async_copy(graph_hbm.at[idx], gra, semg).wait()
      pltpu.make_async_copy(stroke_hbm.at[idx], stv, semg).wait()

    def wait_out(k, b):
      _, _, _, out, _, semo = bufs[b]
      pltpu.make_async_copy(
          out, sum_hbm.at[pl.ds(base + k * CH, CH)], semo).wait()

    def compute(k, b, accs):
      gly, gra, stv, out, _, semo = bufs[b]

      def tgroup(tg, accs):
        accs = list(accs)
        t0 = tg * 4
        for j in range(VPG):
          o = j * 16
          gs = [gly[t0 + dt, pl.ds(o, 16)] for dt in range(4)]
          s1 = [stv[t0 + dt, pl.ds(o, 16)] for dt in range(4)]
          rs = [gra[t0 + dt, pl.ds(o, 16)] for dt in range(4)]
          s2 = [stv[t0 + dt, pl.ds(D_GLYPH + o, 16)] for dt in range(4)]
          for dt in range(4):
            out[t0 + dt, pl.ds(o, 16)] = s1[dt] + gs[dt]
            out[t0 + dt, pl.ds(D_GLYPH + o, 16)] = s2[dt] + rs[dt]
            accs[dt] = accs[dt] + gs[dt] * gs[dt]
        return tuple(accs)

      accs = lax.fori_loop(0, CH // 4, tgroup, accs)
      pltpu.async_copy(out, sum_hbm.at[pl.ds(base + k * CH, CH)], semo)
      return accs

    fire(0, 0)

    def pair(g, accs):
      k0 = 2 * g
      fire(k0 + 1, 1)
      wait_gathers(k0, 0)

      @pl.when(g > 0)
      def _():
        wait_out(k0 - 2, 0)

      accs = compute(k0, 0, accs)

      @pl.when(g < NCH // 2 - 1)
      def _():
        fire(k0 + 2, 0)

      wait_gathers(k0 + 1, 1)

      @pl.when(g > 0)
      def _():
        wait_out(k0 - 1, 1)

      accs = compute(k0 + 1, 1, accs)
      return accs

    accs = lax.fori_loop(0, NCH // 2, pair,
                         (jnp.zeros((16,), jnp.float32),) * 4)
    wait_out(NCH - 2, 0)
    wait_out(NCH - 1, 1)
    sq_v[...] = accs[0] + accs[1] + accs[2] + accs[3]
    pltpu.sync_copy(sq_v, sq_hbm.at[wid])

  return body(ids, glyph, graph, stroke)


_RB = 256
_GRID = HTOK // _RB


def _ln_body(sum_ref, pos_ref, gam_ref, bet_ref, out_ref):
  x = sum_ref[...] + pos_ref[...]
  m = jnp.mean(x, axis=-1, keepdims=True)
  v = jnp.mean((x - m) ** 2, axis=-1, keepdims=True)
  y = (x - m) / jnp.sqrt(v + LN_EPS)
  out_ref[...] = y * gam_ref[...] + bet_ref[...]


def _ln_half(sum_half, pos_table, gamma2, beta2):
  return pl.pallas_call(
      _ln_body,
      grid=(_GRID,),
      in_specs=[
          pl.BlockSpec((_RB, D), lambda i: (i, 0)),
          pl.BlockSpec((_RB, D), lambda i: (i % (SEQ // _RB), 0)),
          pl.BlockSpec((1, D), lambda i: (0, 0)),
          pl.BlockSpec((1, D), lambda i: (0, 0)),
      ],
      out_specs=pl.BlockSpec((_RB, D), lambda i: (i, 0)),
      out_shape=jax.ShapeDtypeStruct((HTOK, D), jnp.float32),
  )(sum_half, pos_table, gamma2, beta2)


def _loss_body(sqa_ref, sqb_ref, loss_ref):
  tot = jnp.sum(sqa_ref[...]) + jnp.sum(sqb_ref[...])
  loss_ref[...] = (tot / float(TOK * D_GLYPH)).reshape(1, 1)


def kernel(input_ids, pos_table, glyph_table, graph_table, stroke_table,
           gamma, beta):
  ids = input_ids.astype(jnp.int32).reshape(TOK)
  gamma2 = gamma.reshape(1, D)
  beta2 = beta.reshape(1, D)

  sum_a, part_a = _sc_gather_sum(
      ids[:HTOK], glyph_table, graph_table, stroke_table)
  sum_b, part_b = _sc_gather_sum(
      ids[HTOK:], glyph_table, graph_table, stroke_table)
  emb_a = _ln_half(sum_a, pos_table, gamma2, beta2)
  emb_b = _ln_half(sum_b, pos_table, gamma2, beta2)

  loss = pl.pallas_call(
      _loss_body,
      out_shape=jax.ShapeDtypeStruct((1, 1), jnp.float32),
  )(part_a, part_b)

  emb = jnp.concatenate([emb_a, emb_b], axis=0)
  return emb.reshape(BATCH, SEQ, D), loss[0, 0]


# speedup vs baseline: 1.2053x; 1.2053x over previous
import functools

import jax
import jax.numpy as jnp
from jax import lax
from jax.experimental import pallas as pl
from jax.experimental.pallas import tpu as pltpu
from jax.experimental.pallas import tpu_sc as plsc

D_GLYPH = 512
D = 1024
BATCH = 4
SEQ = 2048
TOK = BATCH * SEQ
NC = 2
NS = 16
NW = NC * NS
TPW = TOK // NW
CH = 16
NCH = TPW // CH
LN_EPS = 1e-12
VPG = D_GLYPH // 16


def _sc_gather_sum(ids, glyph, graph, stroke):
  mesh = plsc.VectorSubcoreMesh(core_axis_name="c", subcore_axis_name="s")

  @functools.partial(
      pl.kernel,
      mesh=mesh,
      out_type=[
          jax.ShapeDtypeStruct((TOK, D), jnp.float32),
          jax.ShapeDtypeStruct((NW, 16), jnp.float32),
      ],
      scratch_types=[
          pltpu.VMEM((TPW,), jnp.int32),
          pltpu.VMEM((CH, D_GLYPH), jnp.float32),
          pltpu.VMEM((CH, D_GLYPH), jnp.float32),
          pltpu.VMEM((CH, D), jnp.float32),
          pltpu.VMEM((CH, D), jnp.float32),
          pltpu.VMEM((CH, D_GLYPH), jnp.float32),
          pltpu.VMEM((CH, D_GLYPH), jnp.float32),
          pltpu.VMEM((CH, D), jnp.float32),
          pltpu.VMEM((CH, D), jnp.float32),
          pltpu.VMEM((16,), jnp.float32),
          pltpu.SemaphoreType.DMA,
          pltpu.SemaphoreType.DMA,
          pltpu.SemaphoreType.DMA,
          pltpu.SemaphoreType.DMA,
      ],
  )
  def body(ids_hbm, glyph_hbm, graph_hbm, stroke_hbm, sum_hbm, sq_hbm,
           idx_all, gly0, gra0, str0, out0, gly1, gra1, str1, out1,
           sq_v, semg0, semg1, semo0, semo1):
    wid = lax.axis_index("s") * NC + lax.axis_index("c")
    base = wid * TPW
    pltpu.sync_copy(ids_hbm.at[pl.ds(base, TPW)], idx_all)
    bufs = ((gly0, gra0, str0, out0, semg0, semo0),
            (gly1, gra1, str1, out1, semg1, semo1))

    def fire(k, b):
      gly, gra, stv, _, semg, _ = bufs[b]
      idx = idx_all.at[pl.ds(k * CH, CH)]
      pltpu.async_copy(glyph_hbm.at[idx], gly, semg)
      pltpu.async_copy(graph_hbm.at[idx], gra, semg)
      pltpu.async_copy(stroke_hbm.at[idx], stv, semg)

    def wait_gathers(k, b):
      gly, gra, stv, _, semg, _ = bufs[b]
      idx = idx_all.at[pl.ds(k * CH, CH)]
      pltpu.make_async_copy(glyph_hbm.at[idx], gly, semg).wait()
      pltpu.make_async_copy(graph_hbm.at[idx], gra, semg).wait()
      pltpu.make_async_copy(stroke_hbm.at[idx], stv, semg).wait()

    def wait_out(k, b):
      _, _, _, out, _, semo = bufs[b]
      pltpu.make_async_copy(
          out, sum_hbm.at[pl.ds(base + k * CH, CH)], semo).wait()

    def compute(k, b, accs):
      gly, gra, stv, out, _, semo = bufs[b]

      def tgroup(tg, accs):
        accs = list(accs)
        t0 = tg * 4
        for j in range(VPG):
          o = j * 16
          gs = [gly[t0 + dt, pl.ds(o, 16)] for dt in range(4)]
          s1 = [stv[t0 + dt, pl.ds(o, 16)] for dt in range(4)]
          rs = [gra[t0 + dt, pl.ds(o, 16)] for dt in range(4)]
          s2 = [stv[t0 + dt, pl.ds(D_GLYPH + o, 16)] for dt in range(4)]
          for dt in range(4):
            out[t0 + dt, pl.ds(o, 16)] = s1[dt] + gs[dt]
            out[t0 + dt, pl.ds(D_GLYPH + o, 16)] = s2[dt] + rs[dt]
            accs[dt] = accs[dt] + gs[dt] * gs[dt]
        return tuple(accs)

      accs = lax.fori_loop(0, CH // 4, tgroup, accs)
      pltpu.async_copy(out, sum_hbm.at[pl.ds(base + k * CH, CH)], semo)
      return accs

    fire(0, 0)

    def pair(g, accs):
      k0 = 2 * g
      fire(k0 + 1, 1)
      wait_gathers(k0, 0)

      @pl.when(g > 0)
      def _():
        wait_out(k0 - 2, 0)

      accs = compute(k0, 0, accs)

      @pl.when(g < NCH // 2 - 1)
      def _():
        fire(k0 + 2, 0)

      wait_gathers(k0 + 1, 1)

      @pl.when(g > 0)
      def _():
        wait_out(k0 - 1, 1)

      accs = compute(k0 + 1, 1, accs)
      return accs

    accs = lax.fori_loop(0, NCH // 2, pair,
                         (jnp.zeros((16,), jnp.float32),) * 4)
    wait_out(NCH - 2, 0)
    wait_out(NCH - 1, 1)
    sq_v[...] = accs[0] + accs[1] + accs[2] + accs[3]
    pltpu.sync_copy(sq_v, sq_hbm.at[wid])

  return body(ids, glyph, graph, stroke)


_RB = 256
_GRID = TOK // _RB


def _ln_body(sum_ref, pos_ref, gam_ref, bet_ref, sq_ref, out_ref, loss_ref):
  x = sum_ref[...] + pos_ref[...]
  m = jnp.mean(x, axis=-1, keepdims=True)
  q = jnp.mean(x * x, axis=-1, keepdims=True)
  v = q - m * m
  w = jax.lax.rsqrt(v + LN_EPS)
  out_ref[...] = (x - m) * (w * gam_ref[...]) + bet_ref[...]

  @pl.when(pl.program_id(0) == 0)
  def _():
    loss_ref[...] = (jnp.sum(sq_ref[...]) / float(TOK * D_GLYPH)).reshape(1, 1)


def kernel(input_ids, pos_table, glyph_table, graph_table, stroke_table,
           gamma, beta):
  ids = input_ids.astype(jnp.int32).reshape(TOK)
  sum_flat, partials = _sc_gather_sum(
      ids, glyph_table, graph_table, stroke_table)

  emb, loss = pl.pallas_call(
      _ln_body,
      grid=(_GRID,),
      in_specs=[
          pl.BlockSpec((_RB, D), lambda i: (i, 0)),
          pl.BlockSpec((_RB, D), lambda i: (i % (SEQ // _RB), 0)),
          pl.BlockSpec((1, D), lambda i: (0, 0)),
          pl.BlockSpec((1, D), lambda i: (0, 0)),
          pl.BlockSpec((NW, 16), lambda i: (0, 0)),
      ],
      out_specs=[
          pl.BlockSpec((_RB, D), lambda i: (i, 0)),
          pl.BlockSpec((1, 1), lambda i: (0, 0)),
      ],
      out_shape=[
          jax.ShapeDtypeStruct((TOK, D), jnp.float32),
          jax.ShapeDtypeStruct((1, 1), jnp.float32),
      ],
  )(sum_flat, pos_table, gamma.reshape(1, D), beta.reshape(1, D), partials)

  return emb.reshape(BATCH, SEQ, D), loss[0, 0]


# speedup vs baseline: 1.2894x; 1.0698x over previous
import functools

import jax
import jax.numpy as jnp
from jax import lax
from jax.experimental import pallas as pl
from jax.experimental.pallas import tpu as pltpu
from jax.experimental.pallas import tpu_sc as plsc

D_GLYPH = 512
D = 1024
BATCH = 4
SEQ = 2048
TOK = BATCH * SEQ
NC = 2
NS = 16
NW = NC * NS
TPW = TOK // NW
CH = 16
NCH = TPW // CH
LN_EPS = 1e-12
VPG = D_GLYPH // 16


def _sc_gather_sum(ids, glyph, graph, stroke):
  mesh = plsc.VectorSubcoreMesh(core_axis_name="c", subcore_axis_name="s")

  @functools.partial(
      pl.kernel,
      mesh=mesh,
      out_type=[
          jax.ShapeDtypeStruct((TOK, D), jnp.float32),
          jax.ShapeDtypeStruct((NW, 16), jnp.float32),
      ],
      scratch_types=[
          pltpu.VMEM((TPW,), jnp.int32),
          pltpu.VMEM((CH, D_GLYPH), jnp.float32),
          pltpu.VMEM((CH, D_GLYPH), jnp.float32),
          pltpu.VMEM((CH, D), jnp.float32),
          pltpu.VMEM((CH, D), jnp.float32),
          pltpu.VMEM((CH, D_GLYPH), jnp.float32),
          pltpu.VMEM((CH, D_GLYPH), jnp.float32),
          pltpu.VMEM((CH, D), jnp.float32),
          pltpu.VMEM((CH, D), jnp.float32),
          pltpu.VMEM((16,), jnp.float32),
          pltpu.SemaphoreType.DMA,
          pltpu.SemaphoreType.DMA,
          pltpu.SemaphoreType.DMA,
          pltpu.SemaphoreType.DMA,
      ],
  )
  def body(ids_hbm, glyph_hbm, graph_hbm, stroke_hbm, sum_hbm, sq_hbm,
           idx_all, gly0, gra0, str0, out0, gly1, gra1, str1, out1,
           sq_v, semg0, semg1, semo0, semo1):
    wid = lax.axis_index("s") * NC + lax.axis_index("c")
    base = wid * TPW
    pltpu.sync_copy(ids_hbm.at[pl.ds(base, TPW)], idx_all)
    bufs = ((gly0, gra0, str0, out0, semg0, semo0),
            (gly1, gra1, str1, out1, semg1, semo1))

    def fire(k, b):
      gly, gra, stv, _, semg, _ = bufs[b]
      idx = idx_all.at[pl.ds(k * CH, CH)]
      pltpu.async_copy(glyph_hbm.at[idx], gly, semg)
      pltpu.async_copy(graph_hbm.at[idx], gra, semg)
      pltpu.async_copy(stroke_hbm.at[idx], stv, semg)

    def wait_gathers(k, b):
      gly, gra, stv, _, semg, _ = bufs[b]
      idx = idx_all.at[pl.ds(k * CH, CH)]
      pltpu.make_async_copy(glyph_hbm.at[idx], gly, semg).wait()
      pltpu.make_async_copy(graph_hbm.at[idx], gra, semg).wait()
      pltpu.make_async_copy(stroke_hbm.at[idx], stv, semg).wait()

    def wait_out(k, b):
      _, _, _, out, _, semo = bufs[b]
      pltpu.make_async_copy(
          out, sum_hbm.at[pl.ds(base + k * CH, CH)], semo).wait()

    def compute(k, b, accs):
      gly, gra, stv, out, _, semo = bufs[b]

      def tgroup(tg, accs):
        accs = list(accs)
        t0 = tg * 4
        for j in range(VPG):
          o = j * 16
          gs = [gly[t0 + dt, pl.ds(o, 16)] for dt in range(4)]
          s1 = [stv[t0 + dt, pl.ds(o, 16)] for dt in range(4)]
          rs = [gra[t0 + dt, pl.ds(o, 16)] for dt in range(4)]
          s2 = [stv[t0 + dt, pl.ds(D_GLYPH + o, 16)] for dt in range(4)]
          for dt in range(4):
            out[t0 + dt, pl.ds(o, 16)] = s1[dt] + gs[dt]
            out[t0 + dt, pl.ds(D_GLYPH + o, 16)] = s2[dt] + rs[dt]
            accs[dt] = accs[dt] + gs[dt] * gs[dt]
        return tuple(accs)

      accs = lax.fori_loop(0, CH // 4, tgroup, accs)
      pltpu.async_copy(out, sum_hbm.at[pl.ds(base + k * CH, CH)], semo)
      return accs

    fire(0, 0)

    def pair(g, accs):
      k0 = 2 * g
      fire(k0 + 1, 1)
      wait_gathers(k0, 0)

      @pl.when(g > 0)
      def _():
        wait_out(k0 - 2, 0)

      accs = compute(k0, 0, accs)

      @pl.when(g < NCH // 2 - 1)
      def _():
        fire(k0 + 2, 0)

      wait_gathers(k0 + 1, 1)

      @pl.when(g > 0)
      def _():
        wait_out(k0 - 1, 1)

      accs = compute(k0 + 1, 1, accs)
      return accs

    accs = lax.fori_loop(0, NCH // 2, pair,
                         (jnp.zeros((16,), jnp.float32),) * 4)
    wait_out(NCH - 2, 0)
    wait_out(NCH - 1, 1)
    sq_v[...] = accs[0] + accs[1] + accs[2] + accs[3]
    pltpu.sync_copy(sq_v, sq_hbm.at[wid])

  return body(ids, glyph, graph, stroke)


_RB = 512
_GRID = TOK // _RB


def _ln_body(sum_ref, pos_ref, gam_ref, bet_ref, sq_ref, out_ref, loss_ref):
  x = sum_ref[...] + pos_ref[...]
  m = jnp.mean(x, axis=-1, keepdims=True)
  q = jnp.mean(x * x, axis=-1, keepdims=True)
  v = q - m * m
  w = jax.lax.rsqrt(v + LN_EPS)
  out_ref[...] = (x - m) * (w * gam_ref[...]) + bet_ref[...]

  @pl.when(pl.program_id(0) == 0)
  def _():
    loss_ref[...] = (jnp.sum(sq_ref[...]) / float(TOK * D_GLYPH)).reshape(1, 1)


def kernel(input_ids, pos_table, glyph_table, graph_table, stroke_table,
           gamma, beta):
  ids = input_ids.astype(jnp.int32).reshape(TOK)
  sum_flat, partials = _sc_gather_sum(
      ids, glyph_table, graph_table, stroke_table)

  emb, loss = pl.pallas_call(
      _ln_body,
      grid=(_GRID,),
      in_specs=[
          pl.BlockSpec((_RB, D), lambda i: (i, 0)),
          pl.BlockSpec((_RB, D), lambda i: (i % (SEQ // _RB), 0)),
          pl.BlockSpec((1, D), lambda i: (0, 0)),
          pl.BlockSpec((1, D), lambda i: (0, 0)),
          pl.BlockSpec((NW, 16), lambda i: (0, 0)),
      ],
      out_specs=[
          pl.BlockSpec((_RB, D), lambda i: (i, 0)),
          pl.BlockSpec((1, 1), lambda i: (0, 0)),
      ],
      out_shape=[
          jax.ShapeDtypeStruct((TOK, D), jnp.float32),
          jax.ShapeDtypeStruct((1, 1), jnp.float32),
      ],
  )(sum_flat, pos_table, gamma.reshape(1, D), beta.reshape(1, D), partials)

  return emb.reshape(BATCH, SEQ, D), loss[0, 0]


# speedup vs baseline: 1.3300x; 1.0314x over previous
import functools

import jax
import jax.numpy as jnp
from jax import lax
from jax.experimental import pallas as pl
from jax.experimental.pallas import tpu as pltpu
from jax.experimental.pallas import tpu_sc as plsc

D_GLYPH = 512
D = 1024
BATCH = 4
SEQ = 2048
TOK = BATCH * SEQ
NC = 2
NS = 16
NW = NC * NS
TPW = TOK // NW
CH = 16
NCH = TPW // CH
LN_EPS = 1e-12
VPG = D_GLYPH // 16


def _sc_gather_sum(ids, glyph, graph, stroke):
  mesh = plsc.VectorSubcoreMesh(core_axis_name="c", subcore_axis_name="s")

  @functools.partial(
      pl.kernel,
      mesh=mesh,
      out_type=[
          jax.ShapeDtypeStruct((TOK, D), jnp.float32),
          jax.ShapeDtypeStruct((NW, 16), jnp.float32),
      ],
      scratch_types=[
          pltpu.VMEM((TPW,), jnp.int32),
          pltpu.VMEM((CH, D_GLYPH), jnp.float32),
          pltpu.VMEM((CH, D_GLYPH), jnp.float32),
          pltpu.VMEM((CH, D), jnp.float32),
          pltpu.VMEM((CH, D), jnp.float32),
          pltpu.VMEM((CH, D_GLYPH), jnp.float32),
          pltpu.VMEM((CH, D_GLYPH), jnp.float32),
          pltpu.VMEM((CH, D), jnp.float32),
          pltpu.VMEM((CH, D), jnp.float32),
          pltpu.VMEM((16,), jnp.float32),
          pltpu.SemaphoreType.DMA,
          pltpu.SemaphoreType.DMA,
          pltpu.SemaphoreType.DMA,
          pltpu.SemaphoreType.DMA,
      ],
  )
  def body(ids_hbm, glyph_hbm, graph_hbm, stroke_hbm, sum_hbm, sq_hbm,
           idx_all, gly0, gra0, str0, out0, gly1, gra1, str1, out1,
           sq_v, semg0, semg1, semo0, semo1):
    wid = lax.axis_index("s") * NC + lax.axis_index("c")
    base = wid * TPW
    pltpu.sync_copy(ids_hbm.at[pl.ds(base, TPW)], idx_all)
    bufs = ((gly0, gra0, str0, out0, semg0, semo0),
            (gly1, gra1, str1, out1, semg1, semo1))

    def fire(k, b):
      gly, gra, stv, _, semg, _ = bufs[b]
      idx = idx_all.at[pl.ds(k * CH, CH)]
      pltpu.async_copy(glyph_hbm.at[idx], gly, semg)
      pltpu.async_copy(graph_hbm.at[idx], gra, semg)
      pltpu.async_copy(stroke_hbm.at[idx], stv, semg)

    def wait_gathers(k, b):
      gly, gra, stv, _, semg, _ = bufs[b]
      idx = idx_all.at[pl.ds(k * CH, CH)]
      pltpu.make_async_copy(glyph_hbm.at[idx], gly, semg).wait()
      pltpu.make_async_copy(graph_hbm.at[idx], gra, semg).wait()
      pltpu.make_async_copy(stroke_hbm.at[idx], stv, semg).wait()

    def wait_out(k, b):
      _, _, _, out, _, semo = bufs[b]
      pltpu.make_async_copy(
          out, sum_hbm.at[pl.ds(base + k * CH, CH)], semo).wait()

    def compute(k, b, accs):
      gly, gra, stv, out, _, semo = bufs[b]

      def tgroup(tg, accs):
        accs = list(accs)
        t0 = tg * 4
        for j in range(VPG):
          o = j * 16
          gs = [gly[t0 + dt, pl.ds(o, 16)] for dt in range(4)]
          s1 = [stv[t0 + dt, pl.ds(o, 16)] for dt in range(4)]
          rs = [gra[t0 + dt, pl.ds(o, 16)] for dt in range(4)]
          s2 = [stv[t0 + dt, pl.ds(D_GLYPH + o, 16)] for dt in range(4)]
          for dt in range(4):
            out[t0 + dt, pl.ds(o, 16)] = s1[dt] + gs[dt]
            out[t0 + dt, pl.ds(D_GLYPH + o, 16)] = s2[dt] + rs[dt]
            accs[dt] = accs[dt] + gs[dt] * gs[dt]
        return tuple(accs)

      accs = lax.fori_loop(0, CH // 4, tgroup, accs)
      pltpu.async_copy(out, sum_hbm.at[pl.ds(base + k * CH, CH)], semo)
      return accs

    fire(0, 0)

    def pair(g, accs):
      k0 = 2 * g
      fire(k0 + 1, 1)
      wait_gathers(k0, 0)

      @pl.when(g > 0)
      def _():
        wait_out(k0 - 2, 0)

      accs = compute(k0, 0, accs)

      @pl.when(g < NCH // 2 - 1)
      def _():
        fire(k0 + 2, 0)

      wait_gathers(k0 + 1, 1)

      @pl.when(g > 0)
      def _():
        wait_out(k0 - 1, 1)

      accs = compute(k0 + 1, 1, accs)
      return accs

    accs = lax.fori_loop(0, NCH // 2, pair,
                         (jnp.zeros((16,), jnp.float32),) * 4)
    wait_out(NCH - 2, 0)
    wait_out(NCH - 1, 1)
    sq_v[...] = accs[0] + accs[1] + accs[2] + accs[3]
    pltpu.sync_copy(sq_v, sq_hbm.at[wid])

  return body(ids, glyph, graph, stroke)


_RB = 1024
_GRID = TOK // _RB


def _ln_body(sum_ref, pos_ref, gam_ref, bet_ref, sq_ref, out_ref, loss_ref):
  x = sum_ref[...] + pos_ref[...]
  m = jnp.mean(x, axis=-1, keepdims=True)
  q = jnp.mean(x * x, axis=-1, keepdims=True)
  v = q - m * m
  w = jax.lax.rsqrt(v + LN_EPS)
  out_ref[...] = (x - m) * (w * gam_ref[...]) + bet_ref[...]

  @pl.when(pl.program_id(0) == 0)
  def _():
    loss_ref[...] = (jnp.sum(sq_ref[...]) / float(TOK * D_GLYPH)).reshape(1, 1)


def kernel(input_ids, pos_table, glyph_table, graph_table, stroke_table,
           gamma, beta):
  ids = input_ids.astype(jnp.int32).reshape(TOK)
  sum_flat, partials = _sc_gather_sum(
      ids, glyph_table, graph_table, stroke_table)

  emb, loss = pl.pallas_call(
      _ln_body,
      grid=(_GRID,),
      in_specs=[
          pl.BlockSpec((_RB, D), lambda i: (i, 0)),
          pl.BlockSpec((_RB, D), lambda i: (i % (SEQ // _RB), 0)),
          pl.BlockSpec((1, D), lambda i: (0, 0)),
          pl.BlockSpec((1, D), lambda i: (0, 0)),
          pl.BlockSpec((NW, 16), lambda i: (0, 0)),
      ],
      out_specs=[
          pl.BlockSpec((_RB, D), lambda i: (i, 0)),
          pl.BlockSpec((1, 1), lambda i: (0, 0)),
      ],
      out_shape=[
          jax.ShapeDtypeStruct((TOK, D), jnp.float32),
          jax.ShapeDtypeStruct((1, 1), jnp.float32),
      ],
  )(sum_flat, pos_table, gamma.reshape(1, D), beta.reshape(1, D), partials)

  return emb.reshape(BATCH, SEQ, D), loss[0, 0]


# speedup vs baseline: 1.3935x; 1.0478x over previous
import functools

import jax
import jax.numpy as jnp
from jax import lax
from jax.experimental import pallas as pl
from jax.experimental.pallas import tpu as pltpu
from jax.experimental.pallas import tpu_sc as plsc

D_GLYPH = 512
D = 1024
BATCH = 4
SEQ = 2048
TOK = BATCH * SEQ
NC = 2
NS = 16
NW = NC * NS
TPW = TOK // NW
CH = 16
NCH = TPW // CH
LN_EPS = 1e-12
VPG = D_GLYPH // 16


def _sc_gather_sum(ids, glyph, graph, stroke):
  mesh = plsc.VectorSubcoreMesh(core_axis_name="c", subcore_axis_name="s")

  @functools.partial(
      pl.kernel,
      mesh=mesh,
      out_type=[
          jax.ShapeDtypeStruct((TOK, D), jnp.float32),
          jax.ShapeDtypeStruct((NW, 16), jnp.float32),
      ],
      scratch_types=[
          pltpu.VMEM((TPW,), jnp.int32),
          pltpu.VMEM((CH, D_GLYPH), jnp.float32),
          pltpu.VMEM((CH, D_GLYPH), jnp.float32),
          pltpu.VMEM((CH, D), jnp.float32),
          pltpu.VMEM((CH, D), jnp.float32),
          pltpu.VMEM((CH, D_GLYPH), jnp.float32),
          pltpu.VMEM((CH, D_GLYPH), jnp.float32),
          pltpu.VMEM((CH, D), jnp.float32),
          pltpu.VMEM((CH, D), jnp.float32),
          pltpu.VMEM((16,), jnp.float32),
          pltpu.SemaphoreType.DMA,
          pltpu.SemaphoreType.DMA,
          pltpu.SemaphoreType.DMA,
          pltpu.SemaphoreType.DMA,
      ],
  )
  def body(ids_hbm, glyph_hbm, graph_hbm, stroke_hbm, sum_hbm, sq_hbm,
           idx_all, gly0, gra0, str0, out0, gly1, gra1, str1, out1,
           sq_v, semg0, semg1, semo0, semo1):
    wid = lax.axis_index("s") * NC + lax.axis_index("c")
    base = wid * TPW
    pltpu.sync_copy(ids_hbm.at[pl.ds(base, TPW)], idx_all)
    bufs = ((gly0, gra0, str0, out0, semg0, semo0),
            (gly1, gra1, str1, out1, semg1, semo1))

    def fire(k, b):
      gly, gra, stv, _, semg, _ = bufs[b]
      idx = idx_all.at[pl.ds(k * CH, CH)]
      pltpu.async_copy(glyph_hbm.at[idx], gly, semg)
      pltpu.async_copy(graph_hbm.at[idx], gra, semg)
      pltpu.async_copy(stroke_hbm.at[idx], stv, semg)

    def wait_gathers(k, b):
      gly, gra, stv, _, semg, _ = bufs[b]
      idx = idx_all.at[pl.ds(k * CH, CH)]
      pltpu.make_async_copy(glyph_hbm.at[idx], gly, semg).wait()
      pltpu.make_async_copy(graph_hbm.at[idx], gra, semg).wait()
      pltpu.make_async_copy(stroke_hbm.at[idx], stv, semg).wait()

    def wait_out(k, b):
      _, _, _, out, _, semo = bufs[b]
      pltpu.make_async_copy(
          out, sum_hbm.at[pl.ds(base + k * CH, CH)], semo).wait()

    def compute(k, b, accs):
      gly, gra, stv, out, _, semo = bufs[b]

      def tgroup(tg, accs):
        accs = list(accs)
        t0 = tg * 4
        for j in range(VPG):
          o = j * 16
          gs = [gly[t0 + dt, pl.ds(o, 16)] for dt in range(4)]
          s1 = [stv[t0 + dt, pl.ds(o, 16)] for dt in range(4)]
          rs = [gra[t0 + dt, pl.ds(o, 16)] for dt in range(4)]
          s2 = [stv[t0 + dt, pl.ds(D_GLYPH + o, 16)] for dt in range(4)]
          for dt in range(4):
            out[t0 + dt, pl.ds(o, 16)] = s1[dt] + gs[dt]
            out[t0 + dt, pl.ds(D_GLYPH + o, 16)] = s2[dt] + rs[dt]
            accs[dt] = accs[dt] + gs[dt] * gs[dt]
        return tuple(accs)

      accs = lax.fori_loop(0, CH // 4, tgroup, accs)
      pltpu.async_copy(out, sum_hbm.at[pl.ds(base + k * CH, CH)], semo)
      return accs

    fire(0, 0)

    def pair(g, accs):
      k0 = 2 * g
      fire(k0 + 1, 1)
      wait_gathers(k0, 0)

      @pl.when(g > 0)
      def _():
        wait_out(k0 - 2, 0)

      accs = compute(k0, 0, accs)

      @pl.when(g < NCH // 2 - 1)
      def _():
        fire(k0 + 2, 0)

      wait_gathers(k0 + 1, 1)

      @pl.when(g > 0)
      def _():
        wait_out(k0 - 1, 1)

      accs = compute(k0 + 1, 1, accs)
      return accs

    accs = lax.fori_loop(0, NCH // 2, pair,
                         (jnp.zeros((16,), jnp.float32),) * 4)
    wait_out(NCH - 2, 0)
    wait_out(NCH - 1, 1)
    sq_v[...] = accs[0] + accs[1] + accs[2] + accs[3]
    pltpu.sync_copy(sq_v, sq_hbm.at[wid])

  return body(ids, glyph, graph, stroke)


_RB = 2048
_GRID = TOK // _RB


def _ln_body(sum_ref, pos_ref, gam_ref, bet_ref, sq_ref, out_ref, loss_ref):
  x = sum_ref[...] + pos_ref[...]
  m = jnp.mean(x, axis=-1, keepdims=True)
  q = jnp.mean(x * x, axis=-1, keepdims=True)
  v = q - m * m
  w = jax.lax.rsqrt(v + LN_EPS)
  out_ref[...] = (x - m) * (w * gam_ref[...]) + bet_ref[...]

  @pl.when(pl.program_id(0) == 0)
  def _():
    loss_ref[...] = (jnp.sum(sq_ref[...]) / float(TOK * D_GLYPH)).reshape(1, 1)


def kernel(input_ids, pos_table, glyph_table, graph_table, stroke_table,
           gamma, beta):
  ids = input_ids.astype(jnp.int32).reshape(TOK)
  sum_flat, partials = _sc_gather_sum(
      ids, glyph_table, graph_table, stroke_table)

  emb, loss = pl.pallas_call(
      _ln_body,
      grid=(_GRID,),
      in_specs=[
          pl.BlockSpec((_RB, D), lambda i: (i, 0)),
          pl.BlockSpec((_RB, D), lambda i: (i % (SEQ // _RB), 0)),
          pl.BlockSpec((1, D), lambda i: (0, 0)),
          pl.BlockSpec((1, D), lambda i: (0, 0)),
          pl.BlockSpec((NW, 16), lambda i: (0, 0)),
      ],
      out_specs=[
          pl.BlockSpec((_RB, D), lambda i: (i, 0)),
          pl.BlockSpec((1, 1), lambda i: (0, 0)),
      ],
      out_shape=[
          jax.ShapeDtypeStruct((TOK, D), jnp.float32),
          jax.ShapeDtypeStruct((1, 1), jnp.float32),
      ],
  )(sum_flat, pos_table, gamma.reshape(1, D), beta.reshape(1, D), partials)

  return emb.reshape(BATCH, SEQ, D), loss[0, 0]
